# baseline (speedup 1.0000x reference)
import functools

import jax
import jax.numpy as jnp
from jax import lax
from jax.experimental import pallas as pl
from jax.experimental.pallas import tpu as pltpu
from jax.experimental.pallas import tpu_sc as plsc

BATCH = 16384
D = 64
NUM_CORES = 2
NUM_SUBCORES = 16
NUM_WORKERS = NUM_CORES * NUM_SUBCORES
B_PER_W = BATCH // NUM_WORKERS
LANES = 16
CHUNK = 128
N_CHUNKS = B_PER_W // CHUNK
PACKED_ROWS = 500000

_mesh = plsc.VectorSubcoreMesh(core_axis_name="c", subcore_axis_name="s")


@functools.partial(
    pl.kernel,
    mesh=_mesh,
    out_type=jax.ShapeDtypeStruct((BATCH,), jnp.float32),
    scratch_types=[
        pltpu.VMEM((B_PER_W,), jnp.int32),
        pltpu.VMEM((B_PER_W,), jnp.int32),
        pltpu.VMEM((N_CHUNKS, CHUNK), jnp.int32),
        pltpu.VMEM((N_CHUNKS, CHUNK), jnp.int32),
        pltpu.VMEM((CHUNK, 2 * D), jnp.float32),
        pltpu.VMEM((CHUNK, 2 * D), jnp.float32),
        pltpu.VMEM((CHUNK, 2 * D), jnp.float32),
        pltpu.VMEM((CHUNK, 2 * D), jnp.float32),
        pltpu.VMEM((B_PER_W,), jnp.float32),
        pltpu.SemaphoreType.DMA,
        pltpu.SemaphoreType.DMA,
    ],
    compiler_params=pltpu.CompilerParams(needs_layout_passes=False),
)
def _sc_dot(uidx_hbm, midx_hbm, utab2_hbm, mtab2_hbm, out_hbm,
            uidx_v, midx_v, uhalf_v, mhalf_v,
            ubufA, mbufA, ubufB, mbufB, out_v, semA, semB):
    wid = lax.axis_index("s") * NUM_CORES + lax.axis_index("c")
    base = wid * B_PER_W

    pltpu.sync_copy(uidx_hbm.at[wid], uidx_v)
    pltpu.sync_copy(midx_hbm.at[wid], midx_v)

    for c in range(N_CHUNKS):
        for k in range(CHUNK // LANES):
            s = c * CHUNK + k * LANES
            uhalf_v[c, pl.ds(k * LANES, LANES)] = uidx_v[pl.ds(s, LANES)] >> 1
            mhalf_v[c, pl.ds(k * LANES, LANES)] = midx_v[pl.ds(s, LANES)] >> 1

    lane16 = lax.iota(jnp.int32, LANES)

    def issue(c, ubuf, mbuf, sem):
        return [
            pltpu.async_copy(utab2_hbm.at[uhalf_v.at[c]], ubuf, sem),
            pltpu.async_copy(mtab2_hbm.at[mhalf_v.at[c]], mbuf, sem),
        ]

    def compute(c, ubuf, mbuf):
        def group(g, carry):
            off = c * CHUNK + g * LANES
            rowv = g * LANES + lane16
            ucol0 = (uidx_v[pl.ds(off, LANES)] & 1) * D
            mcol0 = (midx_v[pl.ds(off, LANES)] & 1) * D
            acc = jnp.zeros((LANES,), jnp.float32)
            for d in range(D):
                uv = plsc.load_gather(ubuf, [rowv, ucol0 + d])
                mv = plsc.load_gather(mbuf, [rowv, mcol0 + d])
                acc = acc + uv * mv
            out_v[pl.ds(off, LANES)] = acc
            return carry

        lax.fori_loop(0, CHUNK // LANES, group, 0)

    cps = issue(0, ubufA, mbufA, semA)
    for c in range(N_CHUNKS):
        nxt = []
        if c + 1 < N_CHUNKS:
            if c % 2 == 0:
                nxt = issue(c + 1, ubufB, mbufB, semB)
            else:
                nxt = issue(c + 1, ubufA, mbufA, semA)
        for cp in cps:
            cp.wait()
        if c % 2 == 0:
            compute(c, ubufA, mbufA)
        else:
            compute(c, ubufB, mbufB)
        cps = nxt

    pltpu.sync_copy(out_v, out_hbm.at[pl.ds(base, B_PER_W)])


def kernel(userIndices, movieIndices, user_table, movie_table):
    u = userIndices.astype(jnp.int32).reshape(NUM_WORKERS, B_PER_W)
    m = movieIndices.astype(jnp.int32).reshape(NUM_WORKERS, B_PER_W)
    ut2 = user_table.reshape(PACKED_ROWS, 2 * D)
    mt2 = movie_table.reshape(PACKED_ROWS, 2 * D)
    return _sc_dot(u, m, ut2, mt2)


# speedup vs baseline: 2.3204x; 2.3204x over previous
import functools

import jax
import jax.numpy as jnp
from jax import lax
from jax.experimental import pallas as pl
from jax.experimental.pallas import tpu as pltpu
from jax.experimental.pallas import tpu_sc as plsc

BATCH = 16384
D = 64
NUM_CORES = 2
NUM_SUBCORES = 16
NUM_WORKERS = NUM_CORES * NUM_SUBCORES
B_PER_W = BATCH // NUM_WORKERS
LANES = 16
WIN = 128
N_SUB = B_PER_W // 2

_mesh = plsc.VectorSubcoreMesh(core_axis_name="c", subcore_axis_name="s")


@functools.partial(
    pl.kernel,
    mesh=_mesh,
    out_type=jax.ShapeDtypeStruct((BATCH,), jnp.float32),
    scratch_types=[
        pltpu.VMEM((B_PER_W + LANES,), jnp.int32),
        pltpu.VMEM((B_PER_W + LANES,), jnp.int32),
        pltpu.VMEM((D, 2 * WIN), jnp.float32),
        pltpu.VMEM((D, 2 * WIN), jnp.float32),
        pltpu.VMEM((D, 2 * WIN), jnp.float32),
        pltpu.VMEM((D, 2 * WIN), jnp.float32),
        pltpu.VMEM((LANES,), jnp.float32),
        pltpu.VMEM((B_PER_W,), jnp.float32),
        pltpu.SemaphoreType.DMA,
        pltpu.SemaphoreType.DMA,
    ],
    compiler_params=pltpu.CompilerParams(
        needs_layout_passes=False, disable_bounds_checks=True),
)
def _sc_dot(uidx_hbm, midx_hbm, utT_hbm, mtT_hbm, out_hbm,
            uidx_v, midx_v, ubufA, mbufA, ubufB, mbufB,
            shuf, out_v, semA, semB):
    wid = lax.axis_index("s") * NUM_CORES + lax.axis_index("c")
    base = wid * B_PER_W

    pltpu.sync_copy(uidx_hbm.at[wid], uidx_v.at[pl.ds(0, B_PER_W)])
    pltpu.sync_copy(midx_hbm.at[wid], midx_v.at[pl.ds(0, B_PER_W)])

    lane16 = lax.iota(jnp.int32, LANES)
    oct_id = lane16 & 7
    half_id = lane16 >> 3

    def idx_pair(s):
        uv = uidx_v[pl.ds(2 * s, LANES)]
        mv = midx_v[pl.ds(2 * s, LANES)]
        return uv[0], uv[1], mv[0], mv[1]

    def issue(s, ubuf, mbuf, sem):
        u0, u1, m0, m1 = idx_pair(s)
        cps = []
        for k, (u, m) in enumerate(((u0, m0), (u1, m1))):
            su = pl.multiple_of((u >> 7) << 7, WIN)
            sm = pl.multiple_of((m >> 7) << 7, WIN)
            cps.append(pltpu.async_copy(
                utT_hbm.at[pl.ds(0, D), pl.ds(su, WIN)],
                ubuf.at[pl.ds(0, D), pl.ds(k * WIN, WIN)], sem))
            cps.append(pltpu.async_copy(
                mtT_hbm.at[pl.ds(0, D), pl.ds(sm, WIN)],
                mbuf.at[pl.ds(0, D), pl.ds(k * WIN, WIN)], sem))
        return cps

    def compute(s, ubuf, mbuf, acc):
        acc = jnp.where((s & 7) == 0, jnp.zeros((LANES,), jnp.float32), acc)
        u0, u1, m0, m1 = idx_pair(s)
        ucol = (half_id << 7) + jnp.where(
            lane16 < 8, u0 & (WIN - 1), u1 & (WIN - 1))
        mcol = (half_id << 7) + jnp.where(
            lane16 < 8, m0 & (WIN - 1), m1 & (WIN - 1))
        acc2 = jnp.zeros((LANES,), jnp.float32)
        for d0 in range(0, D, 8):
            dvec = d0 + oct_id
            uv = plsc.load_gather(ubuf, [dvec, ucol])
            mv = plsc.load_gather(mbuf, [dvec, mcol])
            acc2 = acc2 + uv * mv
        for step in (1, 2, 4):
            shuf[...] = acc2
            acc2 = acc2 + plsc.load_gather(shuf, [lane16 ^ step])
        p0 = (s & 7) * 2
        acc = jnp.where(lane16 == p0, acc2[0], acc)
        acc = jnp.where(lane16 == p0 + 1, acc2[8], acc)
        out_v[pl.ds(pl.multiple_of((s >> 3) * LANES, LANES), LANES)] = acc
        return acc

    def body(i, acc):
        s0 = 2 * i
        s1 = 2 * i + 1
        cpsA = issue(s0, ubufA, mbufA, semA)
        cpsB = issue(s1, ubufB, mbufB, semB)
        for cp in cpsA:
            cp.wait()
        acc = compute(s0, ubufA, mbufA, acc)
        for cp in cpsB:
            cp.wait()
        acc = compute(s1, ubufB, mbufB, acc)
        return acc

    lax.fori_loop(0, N_SUB // 2, body, jnp.zeros((LANES,), jnp.float32))

    pltpu.sync_copy(out_v, out_hbm.at[pl.ds(base, B_PER_W)])


def kernel(userIndices, movieIndices, user_table, movie_table):
    u = userIndices.astype(jnp.int32).reshape(NUM_WORKERS, B_PER_W)
    m = movieIndices.astype(jnp.int32).reshape(NUM_WORKERS, B_PER_W)
    return _sc_dot(u, m, user_table.T, movie_table.T)


# speedup vs baseline: 2.3864x; 1.0285x over previous
import functools

import jax
import jax.numpy as jnp
from jax import lax
from jax.experimental import pallas as pl
from jax.experimental.pallas import tpu as pltpu
from jax.experimental.pallas import tpu_sc as plsc

BATCH = 16384
D = 64
NUM_CORES = 2
NUM_SUBCORES = 16
NUM_WORKERS = NUM_CORES * NUM_SUBCORES
B_PER_W = BATCH // NUM_WORKERS
LANES = 16
WIN = 128
N_SUB = B_PER_W // 2

_mesh = plsc.VectorSubcoreMesh(core_axis_name="c", subcore_axis_name="s")


@functools.partial(
    pl.kernel,
    mesh=_mesh,
    out_type=jax.ShapeDtypeStruct((BATCH,), jnp.float32),
    scratch_types=[
        pltpu.VMEM((B_PER_W + LANES,), jnp.int32),
        pltpu.VMEM((B_PER_W + LANES,), jnp.int32),
        pltpu.VMEM((D, 2 * WIN), jnp.float32),
        pltpu.VMEM((D, 2 * WIN), jnp.float32),
        pltpu.VMEM((D, 2 * WIN), jnp.float32),
        pltpu.VMEM((D, 2 * WIN), jnp.float32),
        pltpu.VMEM((D, 2 * WIN), jnp.float32),
        pltpu.VMEM((D, 2 * WIN), jnp.float32),
        pltpu.VMEM((LANES,), jnp.float32),
        pltpu.VMEM((B_PER_W,), jnp.float32),
        pltpu.SemaphoreType.DMA,
        pltpu.SemaphoreType.DMA,
        pltpu.SemaphoreType.DMA,
    ],
    compiler_params=pltpu.CompilerParams(
        needs_layout_passes=False, disable_bounds_checks=True),
)
def _sc_dot(uidx_hbm, midx_hbm, utT_hbm, mtT_hbm, out_hbm,
            uidx_v, midx_v, ubufA, mbufA, ubufB, mbufB, ubufC, mbufC,
            shuf, out_v, semA, semB, semC):
    wid = lax.axis_index("s") * NUM_CORES + lax.axis_index("c")
    base = wid * B_PER_W

    pltpu.sync_copy(uidx_hbm.at[wid], uidx_v.at[pl.ds(0, B_PER_W)])
    pltpu.sync_copy(midx_hbm.at[wid], midx_v.at[pl.ds(0, B_PER_W)])

    lane16 = lax.iota(jnp.int32, LANES)
    oct_id = lane16 & 7
    half_id = lane16 >> 3

    def idx_pair(s):
        uv = uidx_v[pl.ds(2 * s, LANES)]
        mv = midx_v[pl.ds(2 * s, LANES)]
        return uv[0], uv[1], mv[0], mv[1]

    def issue(s, ubuf, mbuf, sem):
        u0, u1, m0, m1 = idx_pair(s)
        cps = []
        for k, (u, m) in enumerate(((u0, m0), (u1, m1))):
            su = pl.multiple_of((u >> 7) << 7, WIN)
            sm = pl.multiple_of((m >> 7) << 7, WIN)
            cps.append(pltpu.async_copy(
                utT_hbm.at[pl.ds(0, D), pl.ds(su, WIN)],
                ubuf.at[pl.ds(0, D), pl.ds(k * WIN, WIN)], sem))
            cps.append(pltpu.async_copy(
                mtT_hbm.at[pl.ds(0, D), pl.ds(sm, WIN)],
                mbuf.at[pl.ds(0, D), pl.ds(k * WIN, WIN)], sem))
        return cps

    def compute(s, ubuf, mbuf, acc):
        acc = jnp.where((s & 7) == 0, jnp.zeros((LANES,), jnp.float32), acc)
        u0, u1, m0, m1 = idx_pair(s)
        ucol = (half_id << 7) + jnp.where(
            lane16 < 8, u0 & (WIN - 1), u1 & (WIN - 1))
        mcol = (half_id << 7) + jnp.where(
            lane16 < 8, m0 & (WIN - 1), m1 & (WIN - 1))
        acc2 = jnp.zeros((LANES,), jnp.float32)
        for d0 in range(0, D, 8):
            dvec = d0 + oct_id
            uv = plsc.load_gather(ubuf, [dvec, ucol])
            mv = plsc.load_gather(mbuf, [dvec, mcol])
            acc2 = acc2 + uv * mv
        for step in (1, 2, 4):
            shuf[...] = acc2
            acc2 = acc2 + plsc.load_gather(shuf, [lane16 ^ step])
        p0 = (s & 7) * 2
        acc = jnp.where(lane16 == p0, acc2[0], acc)
        acc = jnp.where(lane16 == p0 + 1, acc2[8], acc)
        out_v[pl.ds(pl.multiple_of((s >> 3) * LANES, LANES), LANES)] = acc
        return acc

    def body(i, acc):
        s0 = 3 * i
        cpsA = issue(s0, ubufA, mbufA, semA)
        cpsB = issue(s0 + 1, ubufB, mbufB, semB)
        cpsC = issue(s0 + 2, ubufC, mbufC, semC)
        for cp in cpsA:
            cp.wait()
        acc = compute(s0, ubufA, mbufA, acc)
        for cp in cpsB:
            cp.wait()
        acc = compute(s0 + 1, ubufB, mbufB, acc)
        for cp in cpsC:
            cp.wait()
        acc = compute(s0 + 2, ubufC, mbufC, acc)
        return acc

    accf = lax.fori_loop(0, N_SUB // 3, body,
                         jnp.zeros((LANES,), jnp.float32))
    last = N_SUB - (N_SUB % 3)
    for cp in issue(last, ubufA, mbufA, semA):
        cp.wait()
    compute(last, ubufA, mbufA, accf)

    pltpu.sync_copy(out_v, out_hbm.at[pl.ds(base, B_PER_W)])


def kernel(userIndices, movieIndices, user_table, movie_table):
    u = userIndices.astype(jnp.int32).reshape(NUM_WORKERS, B_PER_W)
    m = movieIndices.astype(jnp.int32).reshape(NUM_WORKERS, B_PER_W)
    return _sc_dot(u, m, user_table.T, movie_table.T)
